# Initial kernel scaffold; baseline (speedup 1.0000x reference)
import functools

import jax
import jax.numpy as jnp
from jax import lax
from jax.experimental import pallas as pl
from jax.experimental.pallas import tpu as pltpu
from jax.experimental.pallas import tpu_sc as plsc

N = 10000
NP = 10016
F = 16
NC, NS = 2, 16
NW = NC * NS
B = 128
E = 320000
NB = -(-E // (NW * B))
EPW = NB * B
EP = EPW * NW
RPS = NP // NS

_sc_mesh = plsc.VectorSubcoreMesh(core_axis_name="c", subcore_axis_name="s")


@functools.partial(
    pl.kernel,
    out_type=jax.ShapeDtypeStruct((NC, NP, F), jnp.float32),
    mesh=_sc_mesh,
    scratch_types=[
        pltpu.VMEM((NB, B), jnp.int32),
        pltpu.VMEM((B, F), jnp.float32),
        pltpu.VMEM_SHARED((NP, F), jnp.float32),
    ],
)
def _sc_deg(dst_hbm, ones_hbm, zeros_hbm, out_hbm, dstv, ones_v, acc):
    c = lax.axis_index("c")
    s = lax.axis_index("s")
    w = s * NC + c
    pltpu.sync_copy(dst_hbm.at[w], dstv)
    pltpu.sync_copy(ones_hbm, ones_v)
    pltpu.sync_copy(zeros_hbm.at[pl.ds(s * RPS, RPS)],
                    acc.at[pl.ds(s * RPS, RPS)])
    plsc.subcore_barrier()

    def step(j, carry):
        pltpu.sync_copy(ones_v, acc.at[dstv.at[j]], add=True)
        return carry

    lax.fori_loop(0, NB, step, 0)
    plsc.subcore_barrier()
    pltpu.sync_copy(acc.at[pl.ds(s * RPS, RPS)],
                    out_hbm.at[c, pl.ds(s * RPS, RPS)])


@functools.partial(
    pl.kernel,
    out_type=jax.ShapeDtypeStruct((NC, NP, F), jnp.float32),
    mesh=_sc_mesh,
    scratch_types=[
        pltpu.VMEM((NB, B), jnp.int32),
        pltpu.VMEM((NB, B), jnp.int32),
        pltpu.VMEM((B, F), jnp.float32),
        pltpu.VMEM_SHARED((NP, F), jnp.float32),
        pltpu.SemaphoreType.DMA,
    ],
)
def _sc_scatter(src_hbm, dst_hbm, table_hbm, zeros_hbm, out_hbm,
                srcv, dstv, rows, acc, gsem):
    c = lax.axis_index("c")
    s = lax.axis_index("s")
    w = s * NC + c
    pltpu.sync_copy(src_hbm.at[w], srcv)
    pltpu.sync_copy(dst_hbm.at[w], dstv)
    pltpu.sync_copy(zeros_hbm.at[pl.ds(s * RPS, RPS)],
                    acc.at[pl.ds(s * RPS, RPS)])
    plsc.subcore_barrier()

    def step(j, carry):
        pltpu.async_copy(table_hbm.at[srcv.at[j]], rows, gsem).wait()
        pltpu.sync_copy(rows, acc.at[dstv.at[j]], add=True)
        return carry

    lax.fori_loop(0, NB, step, 0)
    plsc.subcore_barrier()
    pltpu.sync_copy(acc.at[pl.ds(s * RPS, RPS)],
                    out_hbm.at[c, pl.ds(s * RPS, RPS)])


def _tc_scale_body(x_ref, w1_ref, degp_ref, y1_ref, dinv_ref):
    deg = degp_ref[0, :, 0:1] + degp_ref[1, :, 0:1] + 1.0
    dinv = jnp.broadcast_to(lax.rsqrt(deg), (NP, F))
    xw = jnp.dot(x_ref[...], w1_ref[...], preferred_element_type=jnp.float32)
    y1_ref[...] = xw * dinv
    dinv_ref[...] = dinv


_tc_scale = pl.pallas_call(
    _tc_scale_body,
    out_shape=(jax.ShapeDtypeStruct((NP, F), jnp.float32),
               jax.ShapeDtypeStruct((NP, F), jnp.float32)),
)


def _tc_mid_body(aggp_ref, y1_ref, dinv_ref, w2p_ref, b1_ref, y2t_ref):
    dinv = dinv_ref[...]
    agg = aggp_ref[0] + aggp_ref[1] + y1_ref[...]
    h1 = jnp.maximum(dinv * agg + b1_ref[...], 0.0)
    rows = lax.broadcasted_iota(jnp.int32, (NP, F), 0)
    h1 = jnp.where(rows < N, h1, 0.0)
    xw2 = jnp.dot(h1, w2p_ref[...], preferred_element_type=jnp.float32)
    y2t_ref[...] = xw2 * dinv


_tc_mid = pl.pallas_call(
    _tc_mid_body,
    out_shape=jax.ShapeDtypeStruct((NP, F), jnp.float32),
)


def _tc_head_body(aggp_ref, y2t_ref, dinv_ref, b2p_ref, wf1p_ref, bf1p_ref,
                  wf2p_ref, bf2_ref, out_ref):
    dinv = dinv_ref[...]
    agg = aggp_ref[0] + aggp_ref[1] + y2t_ref[...]
    h2 = jnp.maximum(dinv * agg + b2p_ref[...], 0.0)
    rows = lax.broadcasted_iota(jnp.int32, (NP, F), 0)
    h2 = jnp.where(rows < N, h2, 0.0)
    h3 = jnp.maximum(
        jnp.dot(h2, wf1p_ref[...], preferred_element_type=jnp.float32)
        + bf1p_ref[...], 0.0)
    h3 = jnp.where(rows < N, h3, 0.0)
    h4 = jnp.dot(h3, wf2p_ref[...], preferred_element_type=jnp.float32)
    total = jnp.sum(h4[:, 0:1]) * (1.0 / N) + bf2_ref[0, 0]
    out_ref[...] = jnp.broadcast_to(total, (1, 1))


_tc_head = pl.pallas_call(
    _tc_head_body,
    out_shape=jax.ShapeDtypeStruct((1, 1), jnp.float32),
)


@jax.jit
def kernel(x, edge_index, W1, b1, W2, b2, Wf1, bf1, Wf2, bf2):
    ei = edge_index.astype(jnp.int32)
    pad = jnp.full((EP - E,), N, jnp.int32)
    src = jnp.concatenate([ei[0], pad]).reshape(NW, NB, B)
    dst = jnp.concatenate([ei[1], pad]).reshape(NW, NB, B)
    x_p = jnp.pad(x, ((0, NP - N), (0, 0)))
    zeros = jnp.zeros((NP, F), jnp.float32)
    ones = jnp.ones((B, F), jnp.float32)

    w2p = jnp.pad(W2, ((0, 0), (0, F - 8)))
    b1p = b1.reshape(1, F)
    b2p = jnp.pad(b2, (0, F - 8)).reshape(1, F)
    wf1p = jnp.pad(Wf1, ((0, F - 8), (0, F - 4)))
    bf1p = jnp.pad(bf1, (0, F - 4)).reshape(1, F)
    wf2p = jnp.pad(Wf2, ((0, F - 4), (0, F - 1)))
    bf2p = bf2.reshape(1, 1)

    deg_parts = _sc_deg(dst, ones, zeros)
    y1, dinv16 = _tc_scale(x_p, W1, deg_parts)
    agg1 = _sc_scatter(src, dst, y1, zeros)
    y2t = _tc_mid(agg1, y1, dinv16, w2p, b1p)
    agg2 = _sc_scatter(src, dst, y2t, zeros)
    out = _tc_head(agg2, y2t, dinv16, b2p, wf1p, bf1p, wf2p, bf2p)
    return out[0, 0]


# speedup vs baseline: 31.7602x; 31.7602x over previous
import functools

import jax
import jax.numpy as jnp
from jax import lax
from jax.experimental import pallas as pl
from jax.experimental.pallas import tpu as pltpu
from jax.experimental.pallas import tpu_sc as plsc

N = 10000
NP = 10112
F = 16
NC, NS = 2, 16
NW = NC * NS
B = 128
E = 320000
NB = -(-E // (NW * B))
EPW = NB * B
EP = EPW * NW
RPS = NP // NS

_sc_mesh = plsc.VectorSubcoreMesh(core_axis_name="c", subcore_axis_name="s")


@functools.partial(
    pl.kernel,
    out_type=jax.ShapeDtypeStruct((NC, NP, F), jnp.float32),
    mesh=_sc_mesh,
    compiler_params=pltpu.CompilerParams(use_tc_tiling_on_sc=False),
    scratch_types=[
        pltpu.VMEM((NB, B), jnp.int32),
        pltpu.VMEM((B, F), jnp.float32),
        pltpu.VMEM_SHARED((NP, F), jnp.float32),
    ],
)
def _sc_deg(dst_hbm, ones_hbm, zeros_hbm, out_hbm, dstv, ones_v, acc):
    c = lax.axis_index("c")
    s = lax.axis_index("s")
    w = s * NC + c
    pltpu.sync_copy(dst_hbm.at[w], dstv)
    pltpu.sync_copy(ones_hbm, ones_v)
    pltpu.sync_copy(zeros_hbm.at[pl.ds(s * RPS, RPS)],
                    acc.at[pl.ds(s * RPS, RPS)])
    plsc.subcore_barrier()

    def step(j, carry):
        pltpu.sync_copy(ones_v, acc.at[dstv.at[j]], add=True)
        return carry

    lax.fori_loop(0, NB, step, 0)
    plsc.subcore_barrier()
    pltpu.sync_copy(acc.at[pl.ds(s * RPS, RPS)],
                    out_hbm.at[c, pl.ds(s * RPS, RPS)])


@functools.partial(
    pl.kernel,
    out_type=jax.ShapeDtypeStruct((NC, NP, F), jnp.float32),
    mesh=_sc_mesh,
    compiler_params=pltpu.CompilerParams(use_tc_tiling_on_sc=False),
    scratch_types=[
        pltpu.VMEM((NB, B), jnp.int32),
        pltpu.VMEM((NB, B), jnp.int32),
        pltpu.VMEM((B, F), jnp.float32),
        pltpu.VMEM_SHARED((NP, F), jnp.float32),
        pltpu.SemaphoreType.DMA,
    ],
)
def _sc_scatter(src_hbm, dst_hbm, table_hbm, zeros_hbm, out_hbm,
                srcv, dstv, rows, acc, gsem):
    c = lax.axis_index("c")
    s = lax.axis_index("s")
    w = s * NC + c
    pltpu.sync_copy(src_hbm.at[w], srcv)
    pltpu.sync_copy(dst_hbm.at[w], dstv)
    pltpu.sync_copy(zeros_hbm.at[pl.ds(s * RPS, RPS)],
                    acc.at[pl.ds(s * RPS, RPS)])
    plsc.subcore_barrier()

    def step(j, carry):
        pltpu.async_copy(table_hbm.at[srcv.at[j]], rows, gsem).wait()
        pltpu.sync_copy(rows, acc.at[dstv.at[j]], add=True)
        return carry

    lax.fori_loop(0, NB, step, 0)
    plsc.subcore_barrier()
    pltpu.sync_copy(acc.at[pl.ds(s * RPS, RPS)],
                    out_hbm.at[c, pl.ds(s * RPS, RPS)])


def _tc_scale_body(x_ref, w1_ref, degp_ref, y1_ref, dinv_ref):
    deg = degp_ref[0, :, 0:1] + degp_ref[1, :, 0:1] + 1.0
    dinv = jnp.broadcast_to(lax.rsqrt(deg), (NP, F))
    xw = jnp.dot(x_ref[...], w1_ref[...], preferred_element_type=jnp.float32)
    y1_ref[...] = xw * dinv
    dinv_ref[...] = dinv


_tc_scale = pl.pallas_call(
    _tc_scale_body,
    out_shape=(jax.ShapeDtypeStruct((NP, F), jnp.float32),
               jax.ShapeDtypeStruct((NP, F), jnp.float32)),
)


def _tc_mid_body(aggp_ref, y1_ref, dinv_ref, w2p_ref, b1_ref, y2t_ref):
    dinv = dinv_ref[...]
    agg = aggp_ref[0] + aggp_ref[1] + y1_ref[...]
    h1 = jnp.maximum(dinv * agg + b1_ref[...], 0.0)
    rows = lax.broadcasted_iota(jnp.int32, (NP, F), 0)
    h1 = jnp.where(rows < N, h1, 0.0)
    xw2 = jnp.dot(h1, w2p_ref[...], preferred_element_type=jnp.float32)
    y2t_ref[...] = xw2 * dinv


_tc_mid = pl.pallas_call(
    _tc_mid_body,
    out_shape=jax.ShapeDtypeStruct((NP, F), jnp.float32),
)


def _tc_head_body(aggp_ref, y2t_ref, dinv_ref, b2p_ref, wf1p_ref, bf1p_ref,
                  wf2p_ref, bf2_ref, out_ref):
    dinv = dinv_ref[...]
    agg = aggp_ref[0] + aggp_ref[1] + y2t_ref[...]
    h2 = jnp.maximum(dinv * agg + b2p_ref[...], 0.0)
    rows = lax.broadcasted_iota(jnp.int32, (NP, F), 0)
    h2 = jnp.where(rows < N, h2, 0.0)
    h3 = jnp.maximum(
        jnp.dot(h2, wf1p_ref[...], preferred_element_type=jnp.float32)
        + bf1p_ref[...], 0.0)
    h3 = jnp.where(rows < N, h3, 0.0)
    h4 = jnp.dot(h3, wf2p_ref[...], preferred_element_type=jnp.float32)
    total = jnp.sum(h4[:, 0:1]) * (1.0 / N) + bf2_ref[0, 0]
    out_ref[...] = jnp.broadcast_to(total, (1, 1))


_tc_head = pl.pallas_call(
    _tc_head_body,
    out_shape=jax.ShapeDtypeStruct((1, 1), jnp.float32),
)


@jax.jit
def kernel(x, edge_index, W1, b1, W2, b2, Wf1, bf1, Wf2, bf2):
    ei = edge_index.astype(jnp.int32)
    pad = jnp.full((EP - E,), N, jnp.int32)
    src = jnp.concatenate([ei[0], pad]).reshape(NW, NB, B)
    dst = jnp.concatenate([ei[1], pad]).reshape(NW, NB, B)
    x_p = jnp.pad(x, ((0, NP - N), (0, 0)))
    zeros = jnp.zeros((NP, F), jnp.float32)
    ones = jnp.ones((B, F), jnp.float32)

    w2p = jnp.pad(W2, ((0, 0), (0, F - 8)))
    b1p = b1.reshape(1, F)
    b2p = jnp.pad(b2, (0, F - 8)).reshape(1, F)
    wf1p = jnp.pad(Wf1, ((0, F - 8), (0, F - 4)))
    bf1p = jnp.pad(bf1, (0, F - 4)).reshape(1, F)
    wf2p = jnp.pad(Wf2, ((0, F - 4), (0, F - 1)))
    bf2p = bf2.reshape(1, 1)

    deg_parts = _sc_deg(dst, ones, zeros)
    y1, dinv16 = _tc_scale(x_p, W1, deg_parts)
    agg1 = _sc_scatter(src, dst, y1, zeros)
    y2t = _tc_mid(agg1, y1, dinv16, w2p, b1p)
    agg2 = _sc_scatter(src, dst, y2t, zeros)
    out = _tc_head(agg2, y2t, dinv16, b2p, wf1p, bf1p, wf2p, bf2p)
    return out[0, 0]


# speedup vs baseline: 43.3969x; 1.3664x over previous
import functools

import jax
import jax.numpy as jnp
from jax import lax
from jax.experimental import pallas as pl
from jax.experimental.pallas import tpu as pltpu
from jax.experimental.pallas import tpu_sc as plsc

N = 10000
NP = 10112
F = 16
NC, NS = 2, 16
NW = NC * NS
B = 128
E = 320000
NB = 80
EPW = NB * B
EP = EPW * NW
RPS = NP // NS
RING = 4
BURST = 8

_sc_mesh = plsc.VectorSubcoreMesh(core_axis_name="c", subcore_axis_name="s")


@functools.partial(
    pl.kernel,
    out_type=jax.ShapeDtypeStruct((NC, NP, F), jnp.float32),
    mesh=_sc_mesh,
    compiler_params=pltpu.CompilerParams(use_tc_tiling_on_sc=False),
    scratch_types=[
        pltpu.VMEM((NB, B), jnp.int32),
        pltpu.VMEM((B, F), jnp.float32),
        pltpu.VMEM_SHARED((NP, F), jnp.float32),
        pltpu.SemaphoreType.DMA,
    ],
)
def _sc_deg(dst_hbm, ones_hbm, zeros_hbm, out_hbm, dstv, ones_v, acc, ssem):
    c = lax.axis_index("c")
    s = lax.axis_index("s")
    w = s * NC + c
    pltpu.sync_copy(dst_hbm.at[w], dstv)
    pltpu.sync_copy(ones_hbm, ones_v)
    pltpu.sync_copy(zeros_hbm.at[pl.ds(s * RPS, RPS)],
                    acc.at[pl.ds(s * RPS, RPS)])
    plsc.subcore_barrier()

    def step(i, carry):
        descs = [
            pltpu.async_copy(ones_v, acc.at[dstv.at[i * BURST + b]], ssem,
                             add=True)
            for b in range(BURST)
        ]
        for d in descs:
            d.wait()
        return carry

    lax.fori_loop(0, NB // BURST, step, 0)
    plsc.subcore_barrier()
    pltpu.sync_copy(acc.at[pl.ds(s * RPS, RPS)],
                    out_hbm.at[c, pl.ds(s * RPS, RPS)])


@functools.partial(
    pl.kernel,
    out_type=jax.ShapeDtypeStruct((NC, NP, F), jnp.float32),
    mesh=_sc_mesh,
    compiler_params=pltpu.CompilerParams(use_tc_tiling_on_sc=False),
    scratch_types=[
        pltpu.VMEM((NB, B), jnp.int32),
        pltpu.VMEM((NB, B), jnp.int32),
        [pltpu.VMEM((B, F), jnp.float32) for _ in range(RING)],
        pltpu.VMEM_SHARED((NP, F), jnp.float32),
        [pltpu.SemaphoreType.DMA for _ in range(RING)],
    ],
)
def _sc_scatter(src_hbm, dst_hbm, table_hbm, zeros_hbm, out_hbm,
                srcv, dstv, rows, acc, gsems):
    c = lax.axis_index("c")
    s = lax.axis_index("s")
    w = s * NC + c
    pltpu.sync_copy(src_hbm.at[w], srcv)
    pltpu.sync_copy(dst_hbm.at[w], dstv)
    pltpu.sync_copy(zeros_hbm.at[pl.ds(s * RPS, RPS)],
                    acc.at[pl.ds(s * RPS, RPS)])
    plsc.subcore_barrier()

    for b in range(RING):
        pltpu.async_copy(table_hbm.at[srcv.at[b]], rows[b], gsems[b])

    def step(i, carry):
        for b in range(RING):
            j = i * RING + b
            pltpu.make_async_copy(table_hbm.at[srcv.at[j]], rows[b],
                                  gsems[b]).wait()
            pltpu.sync_copy(rows[b], acc.at[dstv.at[j]], add=True)

            @pl.when(j + RING < NB)
            def _():
                pltpu.async_copy(table_hbm.at[srcv.at[j + RING]], rows[b],
                                 gsems[b])
        return carry

    lax.fori_loop(0, NB // RING, step, 0)
    plsc.subcore_barrier()
    pltpu.sync_copy(acc.at[pl.ds(s * RPS, RPS)],
                    out_hbm.at[c, pl.ds(s * RPS, RPS)])


def _tc_xw1_body(x_ref, w1_ref, xw1_ref):
    xw = jnp.dot(x_ref[...], w1_ref[...], preferred_element_type=jnp.float32)
    xw1_ref[...] = jnp.concatenate(
        [xw, jnp.zeros((NP - N, F), jnp.float32)], axis=0)


_tc_xw1 = pl.pallas_call(
    _tc_xw1_body,
    out_shape=jax.ShapeDtypeStruct((NP, F), jnp.float32),
)


def _tc_scale_body(xw1_ref, degp_ref, y1_ref, dinv_ref):
    deg = degp_ref[0, :, 0:1] + degp_ref[1, :, 0:1] + 1.0
    dinv = jnp.broadcast_to(lax.rsqrt(deg), (NP, F))
    y1_ref[...] = xw1_ref[...] * dinv
    dinv_ref[...] = dinv


_tc_scale = pl.pallas_call(
    _tc_scale_body,
    out_shape=(jax.ShapeDtypeStruct((NP, F), jnp.float32),
               jax.ShapeDtypeStruct((NP, F), jnp.float32)),
)


def _tc_mid_body(aggp_ref, y1_ref, dinv_ref, w2p_ref, b1_ref, y2t_ref):
    dinv = dinv_ref[...]
    agg = aggp_ref[0] + aggp_ref[1] + y1_ref[...]
    h1 = jnp.maximum(dinv * agg + b1_ref[...], 0.0)
    rows = lax.broadcasted_iota(jnp.int32, (NP, F), 0)
    h1 = jnp.where(rows < N, h1, 0.0)
    xw2 = jnp.dot(h1, w2p_ref[...], preferred_element_type=jnp.float32)
    y2t_ref[...] = xw2 * dinv


_tc_mid = pl.pallas_call(
    _tc_mid_body,
    out_shape=jax.ShapeDtypeStruct((NP, F), jnp.float32),
)


def _tc_head_body(aggp_ref, y2t_ref, dinv_ref, b2p_ref, wf1p_ref, bf1p_ref,
                  wf2p_ref, bf2_ref, out_ref):
    dinv = dinv_ref[...]
    agg = aggp_ref[0] + aggp_ref[1] + y2t_ref[...]
    h2 = jnp.maximum(dinv * agg + b2p_ref[...], 0.0)
    rows = lax.broadcasted_iota(jnp.int32, (NP, F), 0)
    h2 = jnp.where(rows < N, h2, 0.0)
    h3 = jnp.maximum(
        jnp.dot(h2, wf1p_ref[...], preferred_element_type=jnp.float32)
        + bf1p_ref[...], 0.0)
    h3 = jnp.where(rows < N, h3, 0.0)
    h4 = jnp.dot(h3, wf2p_ref[...], preferred_element_type=jnp.float32)
    total = jnp.sum(h4[:, 0:1]) * (1.0 / N) + bf2_ref[0, 0]
    out_ref[...] = jnp.broadcast_to(total, (1, 1))


_tc_head = pl.pallas_call(
    _tc_head_body,
    out_shape=jax.ShapeDtypeStruct((1, 1), jnp.float32),
)


@jax.jit
def kernel(x, edge_index, W1, b1, W2, b2, Wf1, bf1, Wf2, bf2):
    ei = edge_index.astype(jnp.int32)
    pad = jnp.full((EP - E,), N, jnp.int32)
    src = jnp.concatenate([ei[0], pad]).reshape(NW, NB, B)
    dst = jnp.concatenate([ei[1], pad]).reshape(NW, NB, B)
    zeros = jnp.zeros((NP, F), jnp.float32)
    ones = jnp.ones((B, F), jnp.float32)

    w2p = jnp.pad(W2, ((0, 0), (0, F - 8)))
    b1p = b1.reshape(1, F)
    b2p = jnp.pad(b2, (0, F - 8)).reshape(1, F)
    wf1p = jnp.pad(Wf1, ((0, F - 8), (0, F - 4)))
    bf1p = jnp.pad(bf1, (0, F - 4)).reshape(1, F)
    wf2p = jnp.pad(Wf2, ((0, F - 4), (0, F - 1)))
    bf2p = bf2.reshape(1, 1)

    xw1 = _tc_xw1(x, W1)
    deg_parts = _sc_deg(dst, ones, zeros)
    y1, dinv16 = _tc_scale(xw1, deg_parts)
    agg1 = _sc_scatter(src, dst, y1, zeros)
    y2t = _tc_mid(agg1, y1, dinv16, w2p, b1p)
    agg2 = _sc_scatter(src, dst, y2t, zeros)
    out = _tc_head(agg2, y2t, dinv16, b2p, wf1p, bf1p, wf2p, bf2p)
    return out[0, 0]


# speedup vs baseline: 44.1837x; 1.0181x over previous
import functools

import jax
import jax.numpy as jnp
from jax import lax
from jax.experimental import pallas as pl
from jax.experimental.pallas import tpu as pltpu
from jax.experimental.pallas import tpu_sc as plsc

N = 10000
NP = 10112
F = 16
NC, NS = 2, 16
NW = NC * NS
B = 128
E = 320000
NB = 80
EPW = NB * B
EP = EPW * NW
RPS = NP // NS
RING = 8
BURST = 8

_sc_mesh = plsc.VectorSubcoreMesh(core_axis_name="c", subcore_axis_name="s")


@functools.partial(
    pl.kernel,
    out_type=jax.ShapeDtypeStruct((NC, NP, F), jnp.float32),
    mesh=_sc_mesh,
    compiler_params=pltpu.CompilerParams(use_tc_tiling_on_sc=False),
    scratch_types=[
        pltpu.VMEM((NB, B), jnp.int32),
        pltpu.VMEM((B, F), jnp.float32),
        pltpu.VMEM_SHARED((NP, F), jnp.float32),
        pltpu.SemaphoreType.DMA,
    ],
)
def _sc_deg(dst_hbm, ones_hbm, zeros_hbm, out_hbm, dstv, ones_v, acc, ssem):
    c = lax.axis_index("c")
    s = lax.axis_index("s")
    w = s * NC + c
    pltpu.sync_copy(dst_hbm.at[w], dstv)
    pltpu.sync_copy(ones_hbm, ones_v)
    pltpu.sync_copy(zeros_hbm.at[pl.ds(s * RPS, RPS)],
                    acc.at[pl.ds(s * RPS, RPS)])
    plsc.subcore_barrier()

    def step(i, carry):
        descs = [
            pltpu.async_copy(ones_v, acc.at[dstv.at[i * BURST + b]], ssem,
                             add=True)
            for b in range(BURST)
        ]
        for d in descs:
            d.wait()
        return carry

    lax.fori_loop(0, NB // BURST, step, 0)
    plsc.subcore_barrier()
    pltpu.sync_copy(acc.at[pl.ds(s * RPS, RPS)],
                    out_hbm.at[c, pl.ds(s * RPS, RPS)])


@functools.partial(
    pl.kernel,
    out_type=jax.ShapeDtypeStruct((NC, NP, F), jnp.float32),
    mesh=_sc_mesh,
    compiler_params=pltpu.CompilerParams(use_tc_tiling_on_sc=False),
    scratch_types=[
        pltpu.VMEM((NB, B), jnp.int32),
        pltpu.VMEM((NB, B), jnp.int32),
        [pltpu.VMEM((B, F), jnp.float32) for _ in range(RING)],
        pltpu.VMEM_SHARED((NP, F), jnp.float32),
        [pltpu.SemaphoreType.DMA for _ in range(RING)],
    ],
)
def _sc_scatter(src_hbm, dst_hbm, table_hbm, zeros_hbm, out_hbm,
                srcv, dstv, rows, acc, gsems):
    c = lax.axis_index("c")
    s = lax.axis_index("s")
    w = s * NC + c
    pltpu.sync_copy(src_hbm.at[w], srcv)
    pltpu.sync_copy(dst_hbm.at[w], dstv)
    pltpu.sync_copy(zeros_hbm.at[pl.ds(s * RPS, RPS)],
                    acc.at[pl.ds(s * RPS, RPS)])
    plsc.subcore_barrier()

    for b in range(RING):
        pltpu.async_copy(table_hbm.at[srcv.at[b]], rows[b], gsems[b])

    def step(i, carry):
        for b in range(RING):
            j = i * RING + b
            pltpu.make_async_copy(table_hbm.at[srcv.at[j]], rows[b],
                                  gsems[b]).wait()
            pltpu.sync_copy(rows[b], acc.at[dstv.at[j]], add=True)

            @pl.when(j + RING < NB)
            def _():
                pltpu.async_copy(table_hbm.at[srcv.at[j + RING]], rows[b],
                                 gsems[b])
        return carry

    lax.fori_loop(0, NB // RING, step, 0)
    plsc.subcore_barrier()
    pltpu.sync_copy(acc.at[pl.ds(s * RPS, RPS)],
                    out_hbm.at[c, pl.ds(s * RPS, RPS)])


def _tc_xw1_body(x_ref, w1_ref, xw1_ref):
    xw = jnp.dot(x_ref[...], w1_ref[...], preferred_element_type=jnp.float32)
    xw1_ref[...] = jnp.concatenate(
        [xw, jnp.zeros((NP - N, F), jnp.float32)], axis=0)


_tc_xw1 = pl.pallas_call(
    _tc_xw1_body,
    out_shape=jax.ShapeDtypeStruct((NP, F), jnp.float32),
)


def _tc_scale_body(xw1_ref, degp_ref, y1_ref, dinv_ref):
    deg = degp_ref[0, :, 0:1] + degp_ref[1, :, 0:1] + 1.0
    dinv = jnp.broadcast_to(lax.rsqrt(deg), (NP, F))
    y1_ref[...] = xw1_ref[...] * dinv
    dinv_ref[...] = dinv


_tc_scale = pl.pallas_call(
    _tc_scale_body,
    out_shape=(jax.ShapeDtypeStruct((NP, F), jnp.float32),
               jax.ShapeDtypeStruct((NP, F), jnp.float32)),
)


def _tc_mid_body(aggp_ref, y1_ref, dinv_ref, w2p_ref, b1_ref, y2t_ref):
    dinv = dinv_ref[...]
    agg = aggp_ref[0] + aggp_ref[1] + y1_ref[...]
    h1 = jnp.maximum(dinv * agg + b1_ref[...], 0.0)
    rows = lax.broadcasted_iota(jnp.int32, (NP, F), 0)
    h1 = jnp.where(rows < N, h1, 0.0)
    xw2 = jnp.dot(h1, w2p_ref[...], preferred_element_type=jnp.float32)
    y2t_ref[...] = xw2 * dinv


_tc_mid = pl.pallas_call(
    _tc_mid_body,
    out_shape=jax.ShapeDtypeStruct((NP, F), jnp.float32),
)


def _tc_head_body(aggp_ref, y2t_ref, dinv_ref, b2p_ref, wf1p_ref, bf1p_ref,
                  wf2p_ref, bf2_ref, out_ref):
    dinv = dinv_ref[...]
    agg = aggp_ref[0] + aggp_ref[1] + y2t_ref[...]
    h2 = jnp.maximum(dinv * agg + b2p_ref[...], 0.0)
    rows = lax.broadcasted_iota(jnp.int32, (NP, F), 0)
    h2 = jnp.where(rows < N, h2, 0.0)
    h3 = jnp.maximum(
        jnp.dot(h2, wf1p_ref[...], preferred_element_type=jnp.float32)
        + bf1p_ref[...], 0.0)
    h3 = jnp.where(rows < N, h3, 0.0)
    h4 = jnp.dot(h3, wf2p_ref[...], preferred_element_type=jnp.float32)
    total = jnp.sum(h4[:, 0:1]) * (1.0 / N) + bf2_ref[0, 0]
    out_ref[...] = jnp.broadcast_to(total, (1, 1))


_tc_head = pl.pallas_call(
    _tc_head_body,
    out_shape=jax.ShapeDtypeStruct((1, 1), jnp.float32),
)


@jax.jit
def kernel(x, edge_index, W1, b1, W2, b2, Wf1, bf1, Wf2, bf2):
    ei = edge_index.astype(jnp.int32)
    pad = jnp.full((EP - E,), N, jnp.int32)
    src = jnp.concatenate([ei[0], pad]).reshape(NW, NB, B)
    dst = jnp.concatenate([ei[1], pad]).reshape(NW, NB, B)
    zeros = jnp.zeros((NP, F), jnp.float32)
    ones = jnp.ones((B, F), jnp.float32)

    w2p = jnp.pad(W2, ((0, 0), (0, F - 8)))
    b1p = b1.reshape(1, F)
    b2p = jnp.pad(b2, (0, F - 8)).reshape(1, F)
    wf1p = jnp.pad(Wf1, ((0, F - 8), (0, F - 4)))
    bf1p = jnp.pad(bf1, (0, F - 4)).reshape(1, F)
    wf2p = jnp.pad(Wf2, ((0, F - 4), (0, F - 1)))
    bf2p = bf2.reshape(1, 1)

    xw1 = _tc_xw1(x, W1)
    deg_parts = _sc_deg(dst, ones, zeros)
    y1, dinv16 = _tc_scale(xw1, deg_parts)
    agg1 = _sc_scatter(src, dst, y1, zeros)
    y2t = _tc_mid(agg1, y1, dinv16, w2p, b1p)
    agg2 = _sc_scatter(src, dst, y2t, zeros)
    out = _tc_head(agg2, y2t, dinv16, b2p, wf1p, bf1p, wf2p, bf2p)
    return out[0, 0]
